# Initial kernel scaffold; baseline (speedup 1.0000x reference)
import jax
import jax.numpy as jnp
from jax.experimental import pallas as pl


def kernel(x, learn, prox_perm, pool_idx, distal_perm, distal_pre, prev_active):
    raise NotImplementedError("write your pallas kernel here")


# speedup vs baseline: 70.5787x; 70.5787x over previous
import functools

import jax
import jax.numpy as jnp
from jax import lax
from jax.experimental import pallas as pl
from jax.experimental.pallas import tpu as pltpu
from jax.experimental.pallas import tpu_sc as plsc

H, W = 128, 256
NCOLS = 1000
NCOLS_PAD = 1024
CELLS = 40
NCELLS = NCOLS * CELLS
NCELLS_PAD = NCOLS_PAD * CELLS
P = 225
SYN = 40
ACT_THRESH = 13.0
PERM_CONN = 0.5
NUM_ACTIVE = 100

_NC = 2
_NS = 16
_NW = _NC * _NS
_COLS_W = NCOLS_PAD // _NW
_CELLS_W = _COLS_W * CELLS
_CHUNK = 320
_NCHUNK = _CELLS_W // _CHUNK


def _sc_body(x_hbm, pool_hbm, prox_hbm, dperm_hbm, dpre_hbm, prev_hbm,
             overlap_hbm, pred_hbm,
             x_v, pool_v, prox_v, prev_v, dperm_v, dpre_v, pred_v, ov_v):
    wid = lax.axis_index("s") * _NC + lax.axis_index("c")
    lanes = lax.iota(jnp.int32, 16)
    ones = jnp.ones((16,), jnp.float32)
    zeros = jnp.zeros((16,), jnp.float32)

    col0 = wid * _COLS_W
    pltpu.sync_copy(x_hbm, x_v)
    pltpu.sync_copy(pool_hbm.at[pl.ds(col0, _COLS_W)], pool_v)
    pltpu.sync_copy(prox_hbm.at[pl.ds(col0, _COLS_W)], prox_v)
    for g in range(_COLS_W // 16):
        rows = lanes + g * 16

        def pbody(j, acc):
            jv = jnp.full((16,), j, jnp.int32)
            idx = plsc.load_gather(pool_v, [rows, jv])
            vals = plsc.load_gather(x_v, [idx])
            perm = plsc.load_gather(prox_v, [rows, jv])
            hit = jnp.logical_and(vals > 0.98, perm >= PERM_CONN)
            return acc + jnp.where(hit, ones, zeros)

        acc = lax.fori_loop(0, P, pbody, zeros)
        gcol = col0 + g * 16 + lanes
        ov_v[pl.ds(g * 16, 16)] = jnp.where(gcol >= NCOLS, -ones, acc)
    pltpu.sync_copy(ov_v, overlap_hbm.at[pl.ds(col0, _COLS_W)])

    pltpu.sync_copy(prev_hbm, prev_v)
    cell0 = wid * _CELLS_W
    for c in range(_NCHUNK):
        pltpu.sync_copy(dperm_hbm.at[pl.ds(cell0 + c * _CHUNK, _CHUNK)], dperm_v)
        pltpu.sync_copy(dpre_hbm.at[pl.ds(cell0 + c * _CHUNK, _CHUNK)], dpre_v)

        def gbody(g, _, c=c):
            rows = lanes + g * 16

            def jbody(j, acc):
                jv = jnp.full((16,), j, jnp.int32)
                pidx = plsc.load_gather(dpre_v, [rows, jv])
                vals = plsc.load_gather(prev_v, [pidx])
                perm = plsc.load_gather(dperm_v, [rows, jv])
                hit = jnp.logical_and(vals > 0.96, perm >= PERM_CONN)
                return acc + jnp.where(hit, ones, zeros)

            acc = lax.fori_loop(0, SYN, jbody, zeros)
            pred16 = jnp.where(acc >= ACT_THRESH, ones, zeros)
            plsc.store_scatter(pred_v, [c * _CHUNK + g * 16 + lanes], pred16)
            return 0

        lax.fori_loop(0, _CHUNK // 16, gbody, 0)
    pltpu.sync_copy(pred_v, pred_hbm.at[pl.ds(cell0, _CELLS_W)])


def _tc_body(ov_ref, pred_ref, act_ref, anom_ref):
    ov = ov_ref[...]
    t_row = lax.broadcasted_iota(jnp.int32, (1, 256), 1).astype(jnp.float32)
    ge = (ov >= t_row).astype(jnp.float32)
    c_vec = jnp.sum(ge, axis=0, keepdims=True)
    t_sel = jnp.where(c_vec >= float(NUM_ACTIVE), t_row, -1.0)
    T = jnp.max(t_sel)
    cnt_gt = jnp.sum(jnp.where(ov > T, 1.0, 0.0))
    k_rem = float(NUM_ACTIVE) - cnt_gt
    eq = (ov == T).astype(jnp.float32)
    ii = lax.broadcasted_iota(jnp.int32, (NCOLS_PAD, NCOLS_PAD), 0)
    jj = lax.broadcasted_iota(jnp.int32, (NCOLS_PAD, NCOLS_PAD), 1)
    tri = (jj < ii).astype(jnp.float32)
    prefix = lax.dot_general(tri, eq, (((1,), (0,)), ((), ())),
                             preferred_element_type=jnp.float32)
    ca = jnp.where(
        jnp.logical_or(ov > T, jnp.logical_and(eq > 0, prefix < k_rem)),
        1.0, 0.0)
    pred = pred_ref[...]
    col_has = (jnp.sum(pred, axis=1, keepdims=True) > 0).astype(jnp.float32)
    burst = ca * (1.0 - col_has)
    act_ref[...] = jnp.clip(pred * ca + burst, 0.0, 1.0)
    anom = jnp.sum(burst) / jnp.maximum(jnp.sum(ca), 1.0)
    anom_ref[...] = anom.reshape(1, 1)


@jax.jit
def kernel(x, learn, prox_perm, pool_idx, distal_perm, distal_pre, prev_active):
    del learn
    x_flat = x.reshape(-1)
    pool_p = jnp.zeros((NCOLS_PAD, P), jnp.int32).at[:NCOLS].set(pool_idx)
    prox_p = jnp.zeros((NCOLS_PAD, P), jnp.float32).at[:NCOLS].set(prox_perm)
    dperm_p = jnp.zeros((NCELLS_PAD, SYN), jnp.float32).at[:NCELLS].set(distal_perm)
    dpre_p = jnp.zeros((NCELLS_PAD, SYN), jnp.int32).at[:NCELLS].set(distal_pre)

    mesh = plsc.VectorSubcoreMesh(core_axis_name="c", subcore_axis_name="s",
                                  num_cores=_NC, num_subcores=_NS)
    sc = pl.kernel(
        _sc_body,
        compiler_params=pltpu.CompilerParams(use_tc_tiling_on_sc=False,
                                             needs_layout_passes=False),
        out_type=(
            jax.ShapeDtypeStruct((NCOLS_PAD,), jnp.float32),
            jax.ShapeDtypeStruct((NCELLS_PAD,), jnp.float32),
        ),
        mesh=mesh,
        scratch_types=[
            pltpu.VMEM((H * W,), jnp.float32),
            pltpu.VMEM((_COLS_W, P), jnp.int32),
            pltpu.VMEM((_COLS_W, P), jnp.float32),
            pltpu.VMEM((NCELLS,), jnp.float32),
            pltpu.VMEM((_CHUNK, SYN), jnp.float32),
            pltpu.VMEM((_CHUNK, SYN), jnp.int32),
            pltpu.VMEM((_CELLS_W,), jnp.float32),
            pltpu.VMEM((_COLS_W,), jnp.float32),
        ],
    )
    overlap, pred = sc(x_flat, pool_p, prox_p, dperm_p, dpre_p, prev_active)

    act, anom = pl.pallas_call(
        _tc_body,
        out_shape=(
            jax.ShapeDtypeStruct((NCOLS_PAD, CELLS), jnp.float32),
            jax.ShapeDtypeStruct((1, 1), jnp.float32),
        ),
    )(overlap.reshape(NCOLS_PAD, 1), pred.reshape(NCOLS_PAD, CELLS))

    active_cells = act[:NCOLS].reshape(-1)
    anomaly = anom[0, 0]
    return active_cells, anomaly


# speedup vs baseline: 87.5798x; 1.2409x over previous
import jax
import jax.numpy as jnp
from jax import lax
from jax.experimental import pallas as pl
from jax.experimental.pallas import tpu as pltpu
from jax.experimental.pallas import tpu_sc as plsc

H, W = 128, 256
NPIX = H * W
NCOLS = 1000
CELLS = 40
NCELLS = NCOLS * CELLS
PREV_PAD = 40192
PREV_WORDS = PREV_PAD // 32
XWORDS = NPIX // 32
P = 225
SYN = 40
ACT_THRESH = 13.0
PERM_CONN = 0.5
NUM_ACTIVE = 100

_NC = 2
_NS = 16
_NW = _NC * _NS
_COLS_W = 32
_CELL_WIN = 1280
_XW_SH = XWORDS // _NS
_PW_SH = 80


def _sc_body(x_hbm, pool_hbm, prox_hbm, dperm_hbm, dpre_hbm, prev_hbm,
             overlap_hbm, pred_hbm,
             share_v, words_v, xbits_v, pbits_v, pool_v, prox_v,
             dperm_v, dpre_v, ov_v, pred_v,
             xbits_sh, pbits_sh, sem_sp, sem_tm):
    sid = lax.axis_index("s")
    wid = sid * _NC + lax.axis_index("c")
    lanes = lax.iota(jnp.int32, 16)
    onesf = jnp.ones((16,), jnp.float32)
    zerosf = jnp.zeros((16,), jnp.float32)
    onesi = jnp.ones((16,), jnp.int32)
    zerosi = jnp.zeros((16,), jnp.int32)

    col0 = jnp.minimum(wid * _COLS_W, NCOLS - _COLS_W)
    cell0 = jnp.minimum((1250 * wid) // 16 * 16, NCELLS - _CELL_WIN)
    cp_pool = pltpu.make_async_copy(pool_hbm.at[pl.ds(col0, _COLS_W)], pool_v, sem_sp)
    cp_prox = pltpu.make_async_copy(prox_hbm.at[pl.ds(col0, _COLS_W)], prox_v, sem_sp)
    cp_perm = pltpu.make_async_copy(dperm_hbm.at[pl.ds(cell0, _CELL_WIN)], dperm_v, sem_tm)
    cp_pre = pltpu.make_async_copy(dpre_hbm.at[pl.ds(cell0, _CELL_WIN)], dpre_v, sem_tm)
    cp_pool.start()
    cp_prox.start()
    cp_perm.start()
    cp_pre.start()

    def pack(src_hbm, elem0, nwords, thresh, dst_sh, word0):
        nel = nwords * 32
        pltpu.sync_copy(src_hbm.at[pl.ds(elem0, nel)], share_v.at[pl.ds(0, nel)])
        for g in range(nwords // 16):
            acc = zerosi
            for j in range(32):
                idx = (g * 16 + lanes) * 32 + j
                vals = plsc.load_gather(share_v, [idx])
                acc = jnp.bitwise_or(
                    acc, jnp.where(vals > thresh, onesi << j, zerosi))
            words_v[pl.ds(g * 16, 16)] = acc
        pltpu.sync_copy(words_v.at[pl.ds(0, nwords)],
                        dst_sh.at[pl.ds(word0, nwords)])

    pack(x_hbm, sid * (_XW_SH * 32), _XW_SH, 0.98, xbits_sh, sid * _XW_SH)
    pw0 = jnp.minimum(sid * _PW_SH, PREV_WORDS - _PW_SH)
    pack(prev_hbm, pw0 * 32, _PW_SH, 0.96, pbits_sh, pw0)
    plsc.subcore_barrier()
    pltpu.sync_copy(xbits_sh, xbits_v)
    pltpu.sync_copy(pbits_sh, pbits_v)

    cp_pool.wait()
    cp_prox.wait()
    for g in range(_COLS_W // 16):
        rows = lanes + g * 16

        def pbody(jb, acc, rows=rows):
            for jj in range(15):
                j = jb * 15 + jj
                jv = jnp.full((16,), 0, jnp.int32) + j
                idx = plsc.load_gather(pool_v, [rows, jv])
                word = plsc.load_gather(xbits_v, [lax.shift_right_logical(idx, 5)])
                bit = lax.shift_right_logical(word, jnp.bitwise_and(idx, 31)) & 1
                perm = plsc.load_gather(prox_v, [rows, jv])
                hit = jnp.logical_and(bit > 0, perm >= PERM_CONN)
                acc = acc + jnp.where(hit, onesf, zerosf)
            return acc

        acc = lax.fori_loop(0, P // 15, pbody, zerosf)
        ov_v[pl.ds(g * 16, 16)] = acc
    pltpu.sync_copy(ov_v, overlap_hbm.at[pl.ds(col0, _COLS_W)])

    cp_perm.wait()
    cp_pre.wait()

    def gbody(g, _):
        rows = lanes + g * 16
        acc = zerosf
        for j in range(SYN):
            jv = jnp.full((16,), 0, jnp.int32) + j
            pidx = plsc.load_gather(dpre_v, [rows, jv])
            word = plsc.load_gather(pbits_v, [lax.shift_right_logical(pidx, 5)])
            bit = lax.shift_right_logical(word, jnp.bitwise_and(pidx, 31)) & 1
            perm = plsc.load_gather(dperm_v, [rows, jv])
            hit = jnp.logical_and(bit > 0, perm >= PERM_CONN)
            acc = acc + jnp.where(hit, onesf, zerosf)
        pred_v[pl.ds(g * 16, 16)] = jnp.where(acc >= ACT_THRESH, onesf, zerosf)
        return 0

    lax.fori_loop(0, _CELL_WIN // 16, gbody, 0)
    pltpu.sync_copy(pred_v, pred_hbm.at[pl.ds(cell0, _CELL_WIN)])


def _tc_body(ov_ref, pred_ref, act_ref, anom_ref):
    ov = ov_ref[...]
    t_row = lax.broadcasted_iota(jnp.int32, (1, 256), 1).astype(jnp.float32)
    ge = (ov >= t_row).astype(jnp.float32)
    c_vec = jnp.sum(ge, axis=0, keepdims=True)
    t_sel = jnp.where(c_vec >= float(NUM_ACTIVE), t_row, -1.0)
    T = jnp.max(t_sel)
    cnt_gt = jnp.sum(jnp.where(ov > T, 1.0, 0.0))
    k_rem = float(NUM_ACTIVE) - cnt_gt
    eq = (ov == T).astype(jnp.float32)
    ii = lax.broadcasted_iota(jnp.int32, (NCOLS, NCOLS), 0)
    jj = lax.broadcasted_iota(jnp.int32, (NCOLS, NCOLS), 1)
    tri = (jj < ii).astype(jnp.float32)
    prefix = lax.dot_general(tri, eq, (((1,), (0,)), ((), ())),
                             preferred_element_type=jnp.float32)
    ca = jnp.where(
        jnp.logical_or(ov > T, jnp.logical_and(eq > 0, prefix < k_rem)),
        1.0, 0.0)
    pred = pred_ref[...]
    col_has = (jnp.sum(pred, axis=1, keepdims=True) > 0).astype(jnp.float32)
    burst = ca * (1.0 - col_has)
    act_ref[...] = jnp.clip(pred * ca + burst, 0.0, 1.0)
    anom = jnp.sum(burst) / jnp.maximum(jnp.sum(ca), 1.0)
    anom_ref[...] = anom.reshape(1, 1)


@jax.jit
def kernel(x, learn, prox_perm, pool_idx, distal_perm, distal_pre, prev_active):
    del learn
    x_flat = x.reshape(-1)
    prev_p = jnp.zeros((PREV_PAD,), jnp.float32).at[:NCELLS].set(prev_active)

    mesh = plsc.VectorSubcoreMesh(core_axis_name="c", subcore_axis_name="s",
                                  num_cores=_NC, num_subcores=_NS)
    sc = pl.kernel(
        _sc_body,
        compiler_params=pltpu.CompilerParams(use_tc_tiling_on_sc=False,
                                             needs_layout_passes=False),
        out_type=(
            jax.ShapeDtypeStruct((NCOLS,), jnp.float32),
            jax.ShapeDtypeStruct((NCELLS,), jnp.float32),
        ),
        mesh=mesh,
        scratch_types=[
            pltpu.VMEM((_PW_SH * 32,), jnp.float32),
            pltpu.VMEM((_PW_SH,), jnp.int32),
            pltpu.VMEM((XWORDS,), jnp.int32),
            pltpu.VMEM((PREV_WORDS,), jnp.int32),
            pltpu.VMEM((_COLS_W, P), jnp.int32),
            pltpu.VMEM((_COLS_W, P), jnp.float32),
            pltpu.VMEM((_CELL_WIN, SYN), jnp.float32),
            pltpu.VMEM((_CELL_WIN, SYN), jnp.int32),
            pltpu.VMEM((_COLS_W,), jnp.float32),
            pltpu.VMEM((_CELL_WIN,), jnp.float32),
            pltpu.VMEM_SHARED((XWORDS,), jnp.int32),
            pltpu.VMEM_SHARED((PREV_WORDS,), jnp.int32),
            pltpu.SemaphoreType.DMA,
            pltpu.SemaphoreType.DMA,
        ],
    )
    overlap, pred = sc(x_flat, pool_idx, prox_perm, distal_perm, distal_pre,
                       prev_p)

    act, anom = pl.pallas_call(
        _tc_body,
        out_shape=(
            jax.ShapeDtypeStruct((NCOLS, CELLS), jnp.float32),
            jax.ShapeDtypeStruct((1, 1), jnp.float32),
        ),
    )(overlap.reshape(NCOLS, 1), pred.reshape(NCOLS, CELLS))

    active_cells = act.reshape(-1)
    anomaly = anom[0, 0]
    return active_cells, anomaly
